# baseline (speedup 1.0000x reference)
import functools
import math

import jax
import jax.numpy as jnp
from jax import lax
from jax.experimental import pallas as pl
from jax.experimental.pallas import tpu as pltpu
from jax.experimental.pallas import tpu_sc as plsc

B, T, D = 4096, 200, 64
PAD = 0
V = 1000000
VPAD = 1048576
NC, NS = 2, 16
NW = NC * NS
BPW = B // NW
NCH = 2
CH = T // NCH
NLANE = 16
ND = D // NLANE

RBL = 2048
NGRID = VPAD // RBL
LBLKS = (V + RBL - 1) // RBL


def _tc_repack(tabT):
    def body(x_ref, e_ref, o_ref):
        y = lax.dot_general(x_ref[...], e_ref[...], (((0,), (0,)), ((), ())),
                            preferred_element_type=jnp.float32)
        o_ref[...] = jnp.concatenate([y, y], axis=1)

    return pl.pallas_call(
        body,
        grid=(NGRID,),
        in_specs=[
            pl.BlockSpec((D, RBL), lambda c: (0, jnp.minimum(c, LBLKS - 1))),
            pl.BlockSpec((D, D), lambda c: (0, 0)),
        ],
        out_specs=pl.BlockSpec((RBL, 2 * D), lambda c: (c, 0)),
        out_shape=jax.ShapeDtypeStruct((VPAD, 2 * D), jnp.float32),
    )(tabT, jnp.eye(D, dtype=jnp.float32))


def _sc_row_sums(tok3, packed):
    mesh = plsc.VectorSubcoreMesh(core_axis_name="c", subcore_axis_name="s")

    @functools.partial(
        pl.kernel,
        mesh=mesh,
        out_type=jax.ShapeDtypeStruct((B, D), jnp.float32),
        scratch_types=[
            pltpu.VMEM((BPW, NCH, CH), jnp.int32),
            pltpu.VMEM((2, T, 2 * D), jnp.float32),
            pltpu.VMEM((BPW, D), jnp.float32),
            pltpu.SemaphoreType.DMA,
            pltpu.SemaphoreType.DMA,
        ],
        compiler_params=pltpu.CompilerParams(use_tc_tiling_on_sc=True),
    )
    def k(tok_hbm, table_hbm, out_hbm, tok_v, rows_v, sums_v, sem0, sem1):
        sems = (sem0, sem1)
        wid = lax.axis_index("s") * NC + lax.axis_index("c")
        base = wid * BPW
        pltpu.sync_copy(tok_hbm.at[pl.ds(base, BPW)], tok_v)

        def issue(i, buf):
            for c in range(NCH):
                pltpu.async_copy(
                    table_hbm.at[tok_v.at[i, c]],
                    rows_v.at[buf, pl.ds(c * CH, CH)],
                    sems[buf],
                )

        def drain(buf):
            pltpu.make_async_copy(
                table_hbm.at[pl.ds(0, T)], rows_v.at[buf], sems[buf]
            ).wait()

        def accumulate(buf, seq):
            def acc_t(t, accs):
                return tuple(
                    accs[d] + rows_v[buf, t, pl.ds(d * NLANE, NLANE)]
                    for d in range(ND)
                )
            accs = lax.fori_loop(
                0, T, acc_t,
                tuple(jnp.zeros((NLANE,), jnp.float32) for _ in range(ND)),
            )
            for d in range(ND):
                sums_v[seq, pl.ds(d * NLANE, NLANE)] = accs[d]

        issue(0, 0)

        def pair_body(i2, carry):
            a = 2 * i2
            issue(a + 1, 1)
            drain(0)
            accumulate(0, a)

            @pl.when(a + 2 < BPW)
            def _():
                issue(a + 2, 0)

            drain(1)
            accumulate(1, a + 1)
            return carry

        lax.fori_loop(0, BPW // 2, pair_body, 0)
        pltpu.sync_copy(sums_v, out_hbm.at[pl.ds(base, BPW)])

    return k(tok3, packed)


def _tc_head(sums, tokens, Wt, b2, g2, be2):
    def body(s_ref, t_ref, w_ref, b_ref, g_ref, be_ref, o_ref):
        tok = t_ref[...]
        cnt = jnp.sum((tok != PAD).astype(jnp.float32), axis=1, keepdims=True)
        cnt = jnp.maximum(cnt, 1.0)
        pooled = s_ref[...] / cnt
        h = jnp.dot(pooled, w_ref[...], preferred_element_type=jnp.float32)
        h = h + b_ref[...]
        mean = jnp.mean(h, axis=-1, keepdims=True)
        var = jnp.mean(jnp.square(h - mean), axis=-1, keepdims=True)
        hn = (h - mean) * lax.rsqrt(var + 1e-5)
        hl = hn * g_ref[...] + be_ref[...]
        o_ref[...] = 0.5 * hl * (1.0 + lax.erf(hl * (1.0 / math.sqrt(2.0))))

    return pl.pallas_call(
        body,
        out_shape=jax.ShapeDtypeStruct((B, D), jnp.float32),
    )(sums, tokens, Wt, b2, g2, be2)


def kernel(prompt_tokens, emb_table, W, b, ln_gamma, ln_beta):
    tokens = prompt_tokens.astype(jnp.int32)
    tok3 = tokens.reshape(B, NCH, CH)
    packed = _tc_repack(emb_table.T)
    sums = _sc_row_sums(tok3, packed)
    return _tc_head(
        sums, tokens, W.T,
        b.reshape(1, D), ln_gamma.reshape(1, D), ln_beta.reshape(1, D),
    )


# speedup vs baseline: 1.3652x; 1.3652x over previous
import functools
import math

import jax
import jax.numpy as jnp
from jax import lax
from jax.experimental import pallas as pl
from jax.experimental.pallas import tpu as pltpu
from jax.experimental.pallas import tpu_sc as plsc

B, T, D = 4096, 200, 64
PAD = 0
V = 1000000
VPAD = 1048576
NC, NS = 2, 16
NW = NC * NS
BPW = B // NW
NCH = 2
CH = T // NCH
NLANE = 16
ND = D // NLANE

RBL = 8192
NGRID = VPAD // RBL
LBLKS = (V + RBL - 1) // RBL


def _tc_repack(tabT):
    def body(x_ref, e_ref, o_ref):
        y = lax.dot_general(x_ref[...], e_ref[...], (((0,), (0,)), ((), ())),
                            preferred_element_type=jnp.float32)
        o_ref[...] = jnp.concatenate([y, y], axis=1)

    return pl.pallas_call(
        body,
        grid=(NGRID,),
        in_specs=[
            pl.BlockSpec((D, RBL), lambda c: (0, jnp.minimum(c, LBLKS - 1))),
            pl.BlockSpec((D, D), lambda c: (0, 0)),
        ],
        out_specs=pl.BlockSpec((RBL, 2 * D), lambda c: (c, 0)),
        out_shape=jax.ShapeDtypeStruct((VPAD, 2 * D), jnp.float32),
        compiler_params=pltpu.CompilerParams(
            dimension_semantics=("arbitrary",),
            vmem_limit_bytes=100 * 1024 * 1024,
        ),
    )(tabT, jnp.eye(D, dtype=jnp.float32))


def _sc_row_sums(tok3, packed):
    mesh = plsc.VectorSubcoreMesh(core_axis_name="c", subcore_axis_name="s")

    @functools.partial(
        pl.kernel,
        mesh=mesh,
        out_type=jax.ShapeDtypeStruct((B, D), jnp.float32),
        scratch_types=[
            pltpu.VMEM((BPW, NCH, CH), jnp.int32),
            pltpu.VMEM((2, T, 2 * D), jnp.float32),
            pltpu.VMEM((BPW, D), jnp.float32),
            pltpu.SemaphoreType.DMA,
            pltpu.SemaphoreType.DMA,
        ],
        compiler_params=pltpu.CompilerParams(use_tc_tiling_on_sc=True),
    )
    def k(tok_hbm, table_hbm, out_hbm, tok_v, rows_v, sums_v, sem0, sem1):
        sems = (sem0, sem1)
        wid = lax.axis_index("s") * NC + lax.axis_index("c")
        base = wid * BPW
        pltpu.sync_copy(tok_hbm.at[pl.ds(base, BPW)], tok_v)

        def issue(i, buf):
            for c in range(NCH):
                pltpu.async_copy(
                    table_hbm.at[tok_v.at[i, c]],
                    rows_v.at[buf, pl.ds(c * CH, CH)],
                    sems[buf],
                )

        def drain(buf):
            pltpu.make_async_copy(
                table_hbm.at[pl.ds(0, T)], rows_v.at[buf], sems[buf]
            ).wait()

        def accumulate(buf, seq):
            def acc_t(t, accs):
                return tuple(
                    accs[d] + rows_v[buf, t, pl.ds(d * NLANE, NLANE)]
                    for d in range(ND)
                )
            accs = lax.fori_loop(
                0, T, acc_t,
                tuple(jnp.zeros((NLANE,), jnp.float32) for _ in range(ND)),
            )
            for d in range(ND):
                sums_v[seq, pl.ds(d * NLANE, NLANE)] = accs[d]

        issue(0, 0)

        def pair_body(i2, carry):
            a = 2 * i2
            issue(a + 1, 1)
            drain(0)
            accumulate(0, a)

            @pl.when(a + 2 < BPW)
            def _():
                issue(a + 2, 0)

            drain(1)
            accumulate(1, a + 1)
            return carry

        lax.fori_loop(0, BPW // 2, pair_body, 0)
        pltpu.sync_copy(sums_v, out_hbm.at[pl.ds(base, BPW)])

    return k(tok3, packed)


def _tc_head(sums, tokens, Wt, b2, g2, be2):
    def body(s_ref, t_ref, w_ref, b_ref, g_ref, be_ref, o_ref):
        tok = t_ref[...]
        cnt = jnp.sum((tok != PAD).astype(jnp.float32), axis=1, keepdims=True)
        cnt = jnp.maximum(cnt, 1.0)
        pooled = s_ref[...] / cnt
        h = jnp.dot(pooled, w_ref[...], preferred_element_type=jnp.float32)
        h = h + b_ref[...]
        mean = jnp.mean(h, axis=-1, keepdims=True)
        var = jnp.mean(jnp.square(h - mean), axis=-1, keepdims=True)
        hn = (h - mean) * lax.rsqrt(var + 1e-5)
        hl = hn * g_ref[...] + be_ref[...]
        o_ref[...] = 0.5 * hl * (1.0 + lax.erf(hl * (1.0 / math.sqrt(2.0))))

    return pl.pallas_call(
        body,
        out_shape=jax.ShapeDtypeStruct((B, D), jnp.float32),
    )(sums, tokens, Wt, b2, g2, be2)


def kernel(prompt_tokens, emb_table, W, b, ln_gamma, ln_beta):
    tokens = prompt_tokens.astype(jnp.int32)
    tok3 = tokens.reshape(B, NCH, CH)
    packed = _tc_repack(emb_table.T)
    sums = _sc_row_sums(tok3, packed)
    return _tc_head(
        sums, tokens, W.T,
        b.reshape(1, D), ln_gamma.reshape(1, D), ln_beta.reshape(1, D),
    )


# speedup vs baseline: 1.6928x; 1.2400x over previous
import functools
import math

import jax
import jax.numpy as jnp
from jax import lax
from jax.experimental import pallas as pl
from jax.experimental.pallas import tpu as pltpu
from jax.experimental.pallas import tpu_sc as plsc

B, T, D = 4096, 200, 64
PAD = 0
V = 1000000
VPAD = 1048576
NC, NS = 2, 16
NW = NC * NS
BPW = B // NW
NCH = 2
CH = T // NCH
NLANE = 16
ND = D // NLANE

H = VPAD // 2
OBL = 4096
NGRID = H // OBL
LBLKS = (V + OBL - 1) // OBL


def _tc_repack(tabT):
    def body(x1_ref, x2_ref, e_ref, o_ref):
        e = e_ref[...]
        dn = (((0,), (0,)), ((), ()))
        y1 = lax.dot_general(x1_ref[...], e, dn,
                             preferred_element_type=jnp.float32)
        y2 = lax.dot_general(x2_ref[...], e, dn,
                             preferred_element_type=jnp.float32)
        o_ref[...] = jnp.concatenate([y1, y2], axis=1)

    return pl.pallas_call(
        body,
        grid=(NGRID,),
        in_specs=[
            pl.BlockSpec((D, OBL), lambda c: (0, c)),
            pl.BlockSpec((D, OBL),
                         lambda c: (0, jnp.minimum(NGRID + c, LBLKS - 1))),
            pl.BlockSpec((D, D), lambda c: (0, 0)),
        ],
        out_specs=pl.BlockSpec((OBL, 2 * D), lambda c: (c, 0)),
        out_shape=jax.ShapeDtypeStruct((H, 2 * D), jnp.float32),
        compiler_params=pltpu.CompilerParams(
            dimension_semantics=("arbitrary",),
            vmem_limit_bytes=100 * 1024 * 1024,
        ),
    )(tabT, tabT, jnp.eye(D, dtype=jnp.float32))


def _sc_row_sums(tok3, packed):
    mesh = plsc.VectorSubcoreMesh(core_axis_name="c", subcore_axis_name="s")

    @functools.partial(
        pl.kernel,
        mesh=mesh,
        out_type=jax.ShapeDtypeStruct((B, D), jnp.float32),
        scratch_types=[
            pltpu.VMEM((BPW, NCH, CH), jnp.int32),
            pltpu.VMEM((2, T, D), jnp.float32),
            pltpu.VMEM((BPW, D), jnp.float32),
            pltpu.SemaphoreType.DMA,
            pltpu.SemaphoreType.DMA,
        ],
        compiler_params=pltpu.CompilerParams(use_tc_tiling_on_sc=False),
    )
    def k(tok_hbm, table_hbm, out_hbm, tok_v, rows_v, sums_v, sem0, sem1):
        sems = (sem0, sem1)
        wid = lax.axis_index("s") * NC + lax.axis_index("c")
        base = wid * BPW
        pltpu.sync_copy(tok_hbm.at[pl.ds(base, BPW)], tok_v)

        def issue(i, buf):
            for c in range(NCH):
                pltpu.async_copy(
                    table_hbm.at[tok_v.at[i, c]],
                    rows_v.at[buf, pl.ds(c * CH, CH)],
                    sems[buf],
                )

        def drain(buf):
            pltpu.make_async_copy(
                table_hbm.at[pl.ds(0, T)], rows_v.at[buf], sems[buf]
            ).wait()

        def accumulate(buf, seq):
            def acc_t(t, accs):
                return tuple(
                    accs[d] + rows_v[buf, t, pl.ds(d * NLANE, NLANE)]
                    for d in range(ND)
                )
            accs = lax.fori_loop(
                0, T, acc_t,
                tuple(jnp.zeros((NLANE,), jnp.float32) for _ in range(ND)),
            )
            for d in range(ND):
                sums_v[seq, pl.ds(d * NLANE, NLANE)] = accs[d]

        issue(0, 0)

        def pair_body(i2, carry):
            a = 2 * i2
            issue(a + 1, 1)
            drain(0)
            accumulate(0, a)

            @pl.when(a + 2 < BPW)
            def _():
                issue(a + 2, 0)

            drain(1)
            accumulate(1, a + 1)
            return carry

        lax.fori_loop(0, BPW // 2, pair_body, 0)
        pltpu.sync_copy(sums_v, out_hbm.at[pl.ds(base, BPW)])

    return k(tok3, packed)


def _tc_head(sums, tokens, Wt, b2, g2, be2):
    def body(s_ref, t_ref, w_ref, b_ref, g_ref, be_ref, o_ref):
        tok = t_ref[...]
        cnt = jnp.sum((tok != PAD).astype(jnp.float32), axis=1, keepdims=True)
        cnt = jnp.maximum(cnt, 1.0)
        pooled = s_ref[...] / cnt
        h = jnp.dot(pooled, w_ref[...], preferred_element_type=jnp.float32)
        h = h + b_ref[...]
        mean = jnp.mean(h, axis=-1, keepdims=True)
        var = jnp.mean(jnp.square(h - mean), axis=-1, keepdims=True)
        hn = (h - mean) * lax.rsqrt(var + 1e-5)
        hl = hn * g_ref[...] + be_ref[...]
        o_ref[...] = 0.5 * hl * (1.0 + lax.erf(hl * (1.0 / math.sqrt(2.0))))

    return pl.pallas_call(
        body,
        out_shape=jax.ShapeDtypeStruct((B, D), jnp.float32),
    )(sums, tokens, Wt, b2, g2, be2)


def kernel(prompt_tokens, emb_table, W, b, ln_gamma, ln_beta):
    tokens = prompt_tokens.astype(jnp.int32)
    wtok = jnp.where(tokens < H, 2 * tokens, 2 * (tokens - H) + 1)
    tok3 = wtok.reshape(B, NCH, CH)
    packed = _tc_repack(emb_table.T).reshape(VPAD, D)
    sums = _sc_row_sums(tok3, packed)
    return _tc_head(
        sums, tokens, W.T,
        b.reshape(1, D), ln_gamma.reshape(1, D), ln_beta.reshape(1, D),
    )


# speedup vs baseline: 1.8335x; 1.0831x over previous
import functools
import math

import jax
import jax.numpy as jnp
from jax import lax
from jax.experimental import pallas as pl
from jax.experimental.pallas import tpu as pltpu
from jax.experimental.pallas import tpu_sc as plsc

B, T, D = 4096, 200, 64
PAD = 0
V = 1000000
VPAD = 1048576
NC, NS = 2, 16
NW = NC * NS
BPW = B // NW
NCH = 2
CH = T // NCH
NLANE = 16
ND = D // NLANE

H = VPAD // 2
OBL = 8192
NGRID = H // OBL
LBLKS = (V + OBL - 1) // OBL


def _tc_repack(tabT):
    def body(x1_ref, x2_ref, e_ref, o_ref):
        e = e_ref[...]
        dn = (((0,), (0,)), ((), ()))
        y1 = lax.dot_general(x1_ref[...], e, dn,
                             preferred_element_type=jnp.float32)
        y2 = lax.dot_general(x2_ref[...], e, dn,
                             preferred_element_type=jnp.float32)
        o_ref[...] = jnp.concatenate([y1, y2], axis=1)

    return pl.pallas_call(
        body,
        grid=(NGRID,),
        in_specs=[
            pl.BlockSpec((D, OBL), lambda c: (0, c)),
            pl.BlockSpec((D, OBL),
                         lambda c: (0, jnp.minimum(NGRID + c, LBLKS - 1))),
            pl.BlockSpec((D, D), lambda c: (0, 0)),
        ],
        out_specs=pl.BlockSpec((OBL, 2 * D), lambda c: (c, 0)),
        out_shape=jax.ShapeDtypeStruct((H, 2 * D), jnp.float32),
        compiler_params=pltpu.CompilerParams(
            dimension_semantics=("arbitrary",),
            vmem_limit_bytes=100 * 1024 * 1024,
        ),
    )(tabT, tabT, jnp.eye(D, dtype=jnp.float32))


def _sc_row_sums(tok3, packed):
    mesh = plsc.VectorSubcoreMesh(core_axis_name="c", subcore_axis_name="s")

    @functools.partial(
        pl.kernel,
        mesh=mesh,
        out_type=jax.ShapeDtypeStruct((B, D), jnp.float32),
        scratch_types=[
            pltpu.VMEM((BPW, NCH, CH), jnp.int32),
            pltpu.VMEM((2, T, D), jnp.float32),
            pltpu.VMEM((BPW, D), jnp.float32),
            pltpu.SemaphoreType.DMA,
            pltpu.SemaphoreType.DMA,
        ],
        compiler_params=pltpu.CompilerParams(use_tc_tiling_on_sc=False),
    )
    def k(tok_hbm, table_hbm, out_hbm, tok_v, rows_v, sums_v, sem0, sem1):
        sems = (sem0, sem1)
        wid = lax.axis_index("s") * NC + lax.axis_index("c")
        base = wid * BPW
        pltpu.sync_copy(tok_hbm.at[pl.ds(base, BPW)], tok_v)

        def issue(i, buf):
            for c in range(NCH):
                pltpu.async_copy(
                    table_hbm.at[tok_v.at[i, c]],
                    rows_v.at[buf, pl.ds(c * CH, CH)],
                    sems[buf],
                )

        def drain(buf):
            pltpu.make_async_copy(
                table_hbm.at[pl.ds(0, T)], rows_v.at[buf], sems[buf]
            ).wait()

        def accumulate(buf, seq):
            def acc_t(t, accs):
                return tuple(
                    accs[d] + rows_v[buf, t, pl.ds(d * NLANE, NLANE)]
                    for d in range(ND)
                )
            accs = lax.fori_loop(
                0, T, acc_t,
                tuple(jnp.zeros((NLANE,), jnp.float32) for _ in range(ND)),
            )
            for d in range(ND):
                sums_v[seq, pl.ds(d * NLANE, NLANE)] = accs[d]

        issue(0, 0)

        def pair_body(i2, carry):
            a = 2 * i2
            issue(a + 1, 1)
            drain(0)
            accumulate(0, a)

            @pl.when(a + 2 < BPW)
            def _():
                issue(a + 2, 0)

            drain(1)
            accumulate(1, a + 1)
            return carry

        lax.fori_loop(0, BPW // 2, pair_body, 0)
        pltpu.sync_copy(sums_v, out_hbm.at[pl.ds(base, BPW)])

    return k(tok3, packed)


def _tc_head(sums, tokens, Wt, b2, g2, be2):
    def body(s_ref, t_ref, w_ref, b_ref, g_ref, be_ref, o_ref):
        tok = t_ref[...]
        cnt = jnp.sum((tok != PAD).astype(jnp.float32), axis=1, keepdims=True)
        cnt = jnp.maximum(cnt, 1.0)
        pooled = s_ref[...] / cnt
        h = jnp.dot(pooled, w_ref[...], preferred_element_type=jnp.float32)
        h = h + b_ref[...]
        mean = jnp.mean(h, axis=-1, keepdims=True)
        var = jnp.mean(jnp.square(h - mean), axis=-1, keepdims=True)
        hn = (h - mean) * lax.rsqrt(var + 1e-5)
        hl = hn * g_ref[...] + be_ref[...]
        o_ref[...] = 0.5 * hl * (1.0 + lax.erf(hl * (1.0 / math.sqrt(2.0))))

    return pl.pallas_call(
        body,
        out_shape=jax.ShapeDtypeStruct((B, D), jnp.float32),
    )(sums, tokens, Wt, b2, g2, be2)


def kernel(prompt_tokens, emb_table, W, b, ln_gamma, ln_beta):
    tokens = prompt_tokens.astype(jnp.int32)
    wtok = jnp.where(tokens < H, 2 * tokens, 2 * (tokens - H) + 1)
    tok3 = wtok.reshape(B, NCH, CH)
    packed = _tc_repack(emb_table.T).reshape(VPAD, D)
    sums = _sc_row_sums(tok3, packed)
    return _tc_head(
        sums, tokens, W.T,
        b.reshape(1, D), ln_gamma.reshape(1, D), ln_beta.reshape(1, D),
    )


# speedup vs baseline: 1.9082x; 1.0407x over previous
import functools
import math

import jax
import jax.numpy as jnp
from jax import lax
from jax.experimental import pallas as pl
from jax.experimental.pallas import tpu as pltpu
from jax.experimental.pallas import tpu_sc as plsc

B, T, D = 4096, 200, 64
PAD = 0
V = 1000000
VPAD = 1048576
NC, NS = 2, 16
NW = NC * NS
BPW = B // NW
NCH = 2
CH = T // NCH
NLANE = 16
ND = D // NLANE

H = VPAD // 2
OBL = 16384
NGRID = H // OBL
LBLKS = (V + OBL - 1) // OBL


def _tc_repack(tabT):
    def body(x1_ref, x2_ref, e_ref, o_ref):
        e = e_ref[...]
        dn = (((0,), (0,)), ((), ()))
        y1 = lax.dot_general(x1_ref[...], e, dn,
                             preferred_element_type=jnp.float32)
        y2 = lax.dot_general(x2_ref[...], e, dn,
                             preferred_element_type=jnp.float32)
        o_ref[...] = jnp.concatenate([y1, y2], axis=1)

    return pl.pallas_call(
        body,
        grid=(NGRID,),
        in_specs=[
            pl.BlockSpec((D, OBL), lambda c: (0, c)),
            pl.BlockSpec((D, OBL),
                         lambda c: (0, jnp.minimum(NGRID + c, LBLKS - 1))),
            pl.BlockSpec((D, D), lambda c: (0, 0)),
        ],
        out_specs=pl.BlockSpec((OBL, 2 * D), lambda c: (c, 0)),
        out_shape=jax.ShapeDtypeStruct((H, 2 * D), jnp.float32),
        compiler_params=pltpu.CompilerParams(
            dimension_semantics=("arbitrary",),
            vmem_limit_bytes=100 * 1024 * 1024,
        ),
    )(tabT, tabT, jnp.eye(D, dtype=jnp.float32))


def _sc_row_sums(tok3, packed):
    mesh = plsc.VectorSubcoreMesh(core_axis_name="c", subcore_axis_name="s")

    @functools.partial(
        pl.kernel,
        mesh=mesh,
        out_type=jax.ShapeDtypeStruct((B, D), jnp.float32),
        scratch_types=[
            pltpu.VMEM((BPW, NCH, CH), jnp.int32),
            pltpu.VMEM((2, T, D), jnp.float32),
            pltpu.VMEM((BPW, D), jnp.float32),
            pltpu.SemaphoreType.DMA,
            pltpu.SemaphoreType.DMA,
        ],
        compiler_params=pltpu.CompilerParams(use_tc_tiling_on_sc=False),
    )
    def k(tok_hbm, table_hbm, out_hbm, tok_v, rows_v, sums_v, sem0, sem1):
        sems = (sem0, sem1)
        wid = lax.axis_index("s") * NC + lax.axis_index("c")
        base = wid * BPW
        pltpu.sync_copy(tok_hbm.at[pl.ds(base, BPW)], tok_v)

        def issue(i, buf):
            for c in range(NCH):
                pltpu.async_copy(
                    table_hbm.at[tok_v.at[i, c]],
                    rows_v.at[buf, pl.ds(c * CH, CH)],
                    sems[buf],
                )

        def drain(buf):
            pltpu.make_async_copy(
                table_hbm.at[pl.ds(0, T)], rows_v.at[buf], sems[buf]
            ).wait()

        def accumulate(buf, seq):
            def acc_t(t, accs):
                return tuple(
                    accs[d] + rows_v[buf, t, pl.ds(d * NLANE, NLANE)]
                    for d in range(ND)
                )
            accs = lax.fori_loop(
                0, T, acc_t,
                tuple(jnp.zeros((NLANE,), jnp.float32) for _ in range(ND)),
            )
            for d in range(ND):
                sums_v[seq, pl.ds(d * NLANE, NLANE)] = accs[d]

        issue(0, 0)

        def pair_body(i2, carry):
            a = 2 * i2
            issue(a + 1, 1)
            drain(0)
            accumulate(0, a)

            @pl.when(a + 2 < BPW)
            def _():
                issue(a + 2, 0)

            drain(1)
            accumulate(1, a + 1)
            return carry

        lax.fori_loop(0, BPW // 2, pair_body, 0)
        pltpu.sync_copy(sums_v, out_hbm.at[pl.ds(base, BPW)])

    return k(tok3, packed)


def _tc_head(sums, tokens, Wt, b2, g2, be2):
    def body(s_ref, t_ref, w_ref, b_ref, g_ref, be_ref, o_ref):
        tok = t_ref[...]
        cnt = jnp.sum((tok != PAD).astype(jnp.float32), axis=1, keepdims=True)
        cnt = jnp.maximum(cnt, 1.0)
        pooled = s_ref[...] / cnt
        h = jnp.dot(pooled, w_ref[...], preferred_element_type=jnp.float32)
        h = h + b_ref[...]
        mean = jnp.mean(h, axis=-1, keepdims=True)
        var = jnp.mean(jnp.square(h - mean), axis=-1, keepdims=True)
        hn = (h - mean) * lax.rsqrt(var + 1e-5)
        hl = hn * g_ref[...] + be_ref[...]
        o_ref[...] = 0.5 * hl * (1.0 + lax.erf(hl * (1.0 / math.sqrt(2.0))))

    return pl.pallas_call(
        body,
        out_shape=jax.ShapeDtypeStruct((B, D), jnp.float32),
    )(sums, tokens, Wt, b2, g2, be2)


def kernel(prompt_tokens, emb_table, W, b, ln_gamma, ln_beta):
    tokens = prompt_tokens.astype(jnp.int32)
    wtok = jnp.where(tokens < H, 2 * tokens, 2 * (tokens - H) + 1)
    tok3 = wtok.reshape(B, NCH, CH)
    packed = _tc_repack(emb_table.T).reshape(VPAD, D)
    sums = _sc_row_sums(tok3, packed)
    return _tc_head(
        sums, tokens, W.T,
        b.reshape(1, D), ln_gamma.reshape(1, D), ln_beta.reshape(1, D),
    )


# speedup vs baseline: 1.9864x; 1.0410x over previous
import functools
import math

import jax
import jax.numpy as jnp
from jax import lax
from jax.experimental import pallas as pl
from jax.experimental.pallas import tpu as pltpu
from jax.experimental.pallas import tpu_sc as plsc

B, T, D = 4096, 200, 64
PAD = 0
V = 1000000
VPAD = 1048576
NC, NS = 2, 16
NW = NC * NS
BPW = B // NW
CHUNKS = (104, 96)
NLANE = 16
ND = D // NLANE

H = VPAD // 2
OBL = 16384
NGRID = H // OBL
LBLKS = (V + OBL - 1) // OBL


def _tc_repack(tabT):
    def body(x1_ref, x2_ref, e_ref, o_ref):
        e = e_ref[...]
        dn = (((0,), (0,)), ((), ()))
        y1 = lax.dot_general(x1_ref[...], e, dn,
                             preferred_element_type=jnp.float32)
        y2 = lax.dot_general(x2_ref[...], e, dn,
                             preferred_element_type=jnp.float32)
        o_ref[...] = jnp.concatenate([y1, y2], axis=1)

    return pl.pallas_call(
        body,
        grid=(NGRID,),
        in_specs=[
            pl.BlockSpec((D, OBL), lambda c: (0, c)),
            pl.BlockSpec((D, OBL),
                         lambda c: (0, jnp.minimum(NGRID + c, LBLKS - 1))),
            pl.BlockSpec((D, D), lambda c: (0, 0)),
        ],
        out_specs=pl.BlockSpec((OBL, 2 * D), lambda c: (c, 0)),
        out_shape=jax.ShapeDtypeStruct((H, 2 * D), jnp.float32),
        compiler_params=pltpu.CompilerParams(
            dimension_semantics=("arbitrary",),
            vmem_limit_bytes=63 * 1024 * 1024,
        ),
    )(tabT, tabT, jnp.eye(D, dtype=jnp.float32))


def _sc_row_sums(tok2, packed):
    mesh = plsc.VectorSubcoreMesh(core_axis_name="c", subcore_axis_name="s")

    @functools.partial(
        pl.kernel,
        mesh=mesh,
        out_type=jax.ShapeDtypeStruct((B, D), jnp.float32),
        scratch_types=[
            pltpu.VMEM((BPW, T), jnp.int32),
            pltpu.VMEM((2, T, D), jnp.float32),
            pltpu.VMEM((BPW, D), jnp.float32),
            pltpu.SemaphoreType.DMA,
            pltpu.SemaphoreType.DMA,
        ],
        compiler_params=pltpu.CompilerParams(use_tc_tiling_on_sc=False),
    )
    def k(tok_hbm, table_hbm, out_hbm, tok_v, rows_v, sums_v, sem0, sem1):
        sems = (sem0, sem1)
        wid = lax.axis_index("s") * NC + lax.axis_index("c")
        base = wid * BPW
        pltpu.sync_copy(tok_hbm.at[pl.ds(base, BPW)], tok_v)

        def issue(i, buf):
            off = 0
            for c in CHUNKS:
                pltpu.async_copy(
                    table_hbm.at[tok_v.at[i, pl.ds(off, c)]],
                    rows_v.at[buf, pl.ds(off, c)],
                    sems[buf],
                )
                off += c

        def drain(buf):
            pltpu.make_async_copy(
                table_hbm.at[pl.ds(0, T)], rows_v.at[buf], sems[buf]
            ).wait()

        def accumulate(buf, seq):
            def acc_t(t, accs):
                return tuple(
                    accs[d] + rows_v[buf, t, pl.ds(d * NLANE, NLANE)]
                    for d in range(ND)
                )
            accs = lax.fori_loop(
                0, T, acc_t,
                tuple(jnp.zeros((NLANE,), jnp.float32) for _ in range(ND)),
            )
            for d in range(ND):
                sums_v[seq, pl.ds(d * NLANE, NLANE)] = accs[d]

        issue(0, 0)

        def pair_body(i2, carry):
            a = 2 * i2
            issue(a + 1, 1)
            drain(0)
            accumulate(0, a)

            @pl.when(a + 2 < BPW)
            def _():
                issue(a + 2, 0)

            drain(1)
            accumulate(1, a + 1)
            return carry

        lax.fori_loop(0, BPW // 2, pair_body, 0)
        pltpu.sync_copy(sums_v, out_hbm.at[pl.ds(base, BPW)])

    return k(tok2, packed)


def _tc_head(sums, tokens, Wt, b2, g2, be2):
    def body(s_ref, t_ref, w_ref, b_ref, g_ref, be_ref, o_ref):
        tok = t_ref[...]
        cnt = jnp.sum((tok != PAD).astype(jnp.float32), axis=1, keepdims=True)
        cnt = jnp.maximum(cnt, 1.0)
        pooled = s_ref[...] / cnt
        h = jnp.dot(pooled, w_ref[...], preferred_element_type=jnp.float32)
        h = h + b_ref[...]
        mean = jnp.mean(h, axis=-1, keepdims=True)
        var = jnp.mean(jnp.square(h - mean), axis=-1, keepdims=True)
        hn = (h - mean) * lax.rsqrt(var + 1e-5)
        hl = hn * g_ref[...] + be_ref[...]
        o_ref[...] = 0.5 * hl * (1.0 + lax.erf(hl * (1.0 / math.sqrt(2.0))))

    return pl.pallas_call(
        body,
        out_shape=jax.ShapeDtypeStruct((B, D), jnp.float32),
    )(sums, tokens, Wt, b2, g2, be2)


def kernel(prompt_tokens, emb_table, W, b, ln_gamma, ln_beta):
    tokens = prompt_tokens.astype(jnp.int32)
    wtok = jnp.where(tokens < H, 2 * tokens, 2 * (tokens - H) + 1)
    packed = _tc_repack(emb_table.T).reshape(VPAD, D)
    sums = _sc_row_sums(wtok, packed)
    return _tc_head(
        sums, tokens, W.T,
        b.reshape(1, D), ln_gamma.reshape(1, D), ln_beta.reshape(1, D),
    )


# speedup vs baseline: 2.1568x; 1.0858x over previous
import functools
import math

import jax
import jax.numpy as jnp
from jax import lax
from jax.experimental import pallas as pl
from jax.experimental.pallas import tpu as pltpu
from jax.experimental.pallas import tpu_sc as plsc

B, T, D = 4096, 200, 64
PAD = 0
V = 1000000
VPAD = 1048576
NC, NS = 2, 16
NW = NC * NS
BPW = B // NW
CHUNKS = (104, 96)
NLANE = 16
ND = D // NLANE

H = VPAD // 2
OBL = 16384
NGRID = H // OBL
LBLKS = (V + OBL - 1) // OBL


def _tc_repack(tabT):
    def body(x1_ref, x2_ref, e_ref, o_ref):
        e = e_ref[...]
        dn = (((0,), (0,)), ((), ()))
        y1 = lax.dot_general(x1_ref[...], e, dn,
                             preferred_element_type=jnp.float32)
        y2 = lax.dot_general(x2_ref[...], e, dn,
                             preferred_element_type=jnp.float32)
        o_ref[...] = jnp.concatenate([y1, y2], axis=1)

    return pl.pallas_call(
        body,
        grid=(NGRID,),
        in_specs=[
            pl.BlockSpec((D, OBL), lambda c: (0, c)),
            pl.BlockSpec((D, OBL),
                         lambda c: (0, jnp.minimum(NGRID + c, LBLKS - 1))),
            pl.BlockSpec((D, D), lambda c: (0, 0)),
        ],
        out_specs=pl.BlockSpec((OBL, 2 * D), lambda c: (c, 0)),
        out_shape=jax.ShapeDtypeStruct((H, 2 * D), jnp.float32),
        compiler_params=pltpu.CompilerParams(
            dimension_semantics=("arbitrary",),
            vmem_limit_bytes=63 * 1024 * 1024,
        ),
    )(tabT, tabT, jnp.eye(D, dtype=jnp.float32))


def _sc_row_sums(tok2, packed):
    mesh = plsc.VectorSubcoreMesh(core_axis_name="c", subcore_axis_name="s")

    @functools.partial(
        pl.kernel,
        mesh=mesh,
        out_type=jax.ShapeDtypeStruct((B, D), jnp.float32),
        scratch_types=[
            pltpu.VMEM((BPW, T), jnp.int32),
            pltpu.VMEM((4, T, D), jnp.float32),
            pltpu.VMEM((BPW, D), jnp.float32),
            pltpu.SemaphoreType.DMA,
            pltpu.SemaphoreType.DMA,
            pltpu.SemaphoreType.DMA,
            pltpu.SemaphoreType.DMA,
        ],
        compiler_params=pltpu.CompilerParams(use_tc_tiling_on_sc=False),
    )
    def k(tok_hbm, table_hbm, out_hbm, tok_v, rows_v, sums_v,
          sem0, sem1, sem2, sem3):
        sems = (sem0, sem1, sem2, sem3)
        wid = lax.axis_index("s") * NC + lax.axis_index("c")
        base = wid * BPW
        pltpu.sync_copy(tok_hbm.at[pl.ds(base, BPW)], tok_v)

        def issue(i, buf):
            off = 0
            for c in CHUNKS:
                pltpu.async_copy(
                    table_hbm.at[tok_v.at[i, pl.ds(off, c)]],
                    rows_v.at[buf, pl.ds(off, c)],
                    sems[buf],
                )
                off += c

        def drain(buf):
            pltpu.make_async_copy(
                table_hbm.at[pl.ds(0, T)], rows_v.at[buf], sems[buf]
            ).wait()

        def accumulate(buf, seq):
            def acc_t(t, accs):
                return tuple(
                    accs[d] + rows_v[buf, t, pl.ds(d * NLANE, NLANE)]
                    for d in range(ND)
                )
            accs = lax.fori_loop(
                0, T, acc_t,
                tuple(jnp.zeros((NLANE,), jnp.float32) for _ in range(ND)),
            )
            for d in range(ND):
                sums_v[seq, pl.ds(d * NLANE, NLANE)] = accs[d]

        for b in range(3):
            issue(b, b)

        def quad_body(i4, carry):
            a = 4 * i4
            for b in range(4):
                drain(b)
                accumulate(b, a + b)

                @pl.when(a + b + 3 < BPW)
                def _():
                    issue(a + b + 3, (b + 3) % 4)
            return carry

        lax.fori_loop(0, BPW // 4, quad_body, 0)
        pltpu.sync_copy(sums_v, out_hbm.at[pl.ds(base, BPW)])

    return k(tok2, packed)


def _tc_head(sums, tokens, Wt, b2, g2, be2):
    def body(s_ref, t_ref, w_ref, b_ref, g_ref, be_ref, o_ref):
        tok = t_ref[...]
        cnt = jnp.sum((tok != PAD).astype(jnp.float32), axis=1, keepdims=True)
        cnt = jnp.maximum(cnt, 1.0)
        pooled = s_ref[...] / cnt
        h = jnp.dot(pooled, w_ref[...], preferred_element_type=jnp.float32)
        h = h + b_ref[...]
        mean = jnp.mean(h, axis=-1, keepdims=True)
        var = jnp.mean(jnp.square(h - mean), axis=-1, keepdims=True)
        hn = (h - mean) * lax.rsqrt(var + 1e-5)
        hl = hn * g_ref[...] + be_ref[...]
        o_ref[...] = 0.5 * hl * (1.0 + lax.erf(hl * (1.0 / math.sqrt(2.0))))

    return pl.pallas_call(
        body,
        out_shape=jax.ShapeDtypeStruct((B, D), jnp.float32),
    )(sums, tokens, Wt, b2, g2, be2)


def kernel(prompt_tokens, emb_table, W, b, ln_gamma, ln_beta):
    tokens = prompt_tokens.astype(jnp.int32)
    wtok = jnp.where(tokens < H, 2 * tokens, 2 * (tokens - H) + 1)
    packed = _tc_repack(emb_table.T).reshape(VPAD, D)
    sums = _sc_row_sums(wtok, packed)
    return _tc_head(
        sums, tokens, W.T,
        b.reshape(1, D), ln_gamma.reshape(1, D), ln_beta.reshape(1, D),
    )
